# baseline (speedup 1.0000x reference)
import functools

import jax
import jax.numpy as jnp
from jax import lax
from jax.experimental import pallas as pl
from jax.experimental.pallas import tpu as pltpu
from jax.experimental.pallas import tpu_sc as plsc

N = 10000
D = 128
H = 64
C = 10

NC = 2
NS = 16
NW = NC * NS

NP = 10240
NR = NP // 2
RPR = NR // NS
TR = 5040
MB = 40
CK = 128
CHD = 82
CHL = 168
EB = 24
CHP = 600
YPAD = 16384
EPD = NW * CHD * CK
EPL = NS * CHL * CK

_mesh = plsc.VectorSubcoreMesh(core_axis_name="c", subcore_axis_name="s")


def _bcast_lane(vec16, j):
    idx = jnp.full((16,), j, dtype=jnp.int32)
    return lax.gather(
        vec16, idx[:, None],
        lax.GatherDimensionNumbers(
            offset_dims=(), collapsed_slice_dims=(0,), start_index_map=(0,)),
        (1,), mode=lax.GatherScatterMode.PROMISE_IN_BOUNDS)


@functools.partial(
    pl.kernel,
    out_type=jax.ShapeDtypeStruct((NC, NP), jnp.float32),
    mesh=_mesh,
    scratch_types=[
        pltpu.VMEM((CHD, CK), jnp.int32),
        pltpu.VMEM((CHD * CK,), jnp.float32),
        pltpu.VMEM_SHARED((NP,), jnp.float32),
    ],
)
def _deg_kernel(dst_hbm, ew_hbm, zz_hbm, out_hbm, dst_v, ew_v, acc_sh):
    cid = lax.axis_index("c")
    sid = lax.axis_index("s")
    wid = sid * NC + cid
    rpt = NP // NS
    pltpu.sync_copy(dst_hbm.at[wid], dst_v)
    pltpu.sync_copy(ew_hbm.at[wid], ew_v)
    pltpu.sync_copy(zz_hbm.at[pl.ds(sid * rpt, rpt)],
                    acc_sh.at[pl.ds(sid * rpt, rpt)])
    plsc.subcore_barrier()

    def body(i, _):
        pltpu.sync_copy(ew_v.at[pl.ds(i * CK, CK)],
                        acc_sh.at[dst_v.at[i]], add=True)
        return ()

    lax.fori_loop(0, CHD, body, ())
    plsc.subcore_barrier()
    pltpu.sync_copy(acc_sh.at[pl.ds(sid * rpt, rpt)],
                    out_hbm.at[cid, pl.ds(sid * rpt, rpt)])


@functools.partial(
    pl.kernel,
    out_type=jax.ShapeDtypeStruct((NR, 2 * H), jnp.float32),
    mesh=_mesh,
    scratch_types=[
        pltpu.VMEM((EB, CK), jnp.int32),
        pltpu.VMEM((EB, CK), jnp.int32),
        pltpu.VMEM((EB * CK,), jnp.float32),
        pltpu.VMEM((CK, 2 * H), jnp.float32),
        pltpu.VMEM((CK, 2 * H), jnp.float32),
        pltpu.VMEM((1, CK), jnp.int32),
        pltpu.VMEM((1, CK), jnp.int32),
        pltpu.VMEM((1, CK), jnp.int32),
        pltpu.VMEM((MB, 2 * H), jnp.float32),
        pltpu.VMEM((2 * RPR,), jnp.float32),
        pltpu.VMEM((H,), jnp.float32),
        pltpu.VMEM_SHARED((NR, 2 * H), jnp.float32),
        pltpu.VMEM_SHARED((TR, 2 * H), jnp.float32),
        pltpu.SemaphoreType.DMA,
        pltpu.SemaphoreType.DMA,
    ],
)
def _gcn_kernel(y_hbm, src_hbm, dst_hbm, ew_hbm, dinv_hbm, b1_hbm,
                out_hbm, src_v, dst_v, ew_v, buf_v, buf2_v, gi_v, gi2_v,
                si_v, abuf_v, dinv_v, b1_v, acc_sh, ytbl_sh, sem0, sem1):
    cid = lax.axis_index("c")
    sid = lax.axis_index("s")
    pltpu.sync_copy(dinv_hbm.at[pl.ds(sid * 2 * RPR, 2 * RPR)], dinv_v)
    pltpu.sync_copy(b1_hbm, b1_v)

    def zero_buf(_i, _):
        for q in range(2 * H // 16):
            buf_v[_i, pl.ds(q * 16, 16)] = jnp.zeros((16,), jnp.float32)
        return ()

    lax.fori_loop(0, CK, zero_buf, ())
    for k in range(RPR // CK):
        pltpu.sync_copy(buf_v, acc_sh.at[pl.ds(sid * RPR + k * CK, CK)])
    pltpu.sync_copy(buf_v.at[pl.ds(0, RPR % CK)],
                    acc_sh.at[pl.ds(sid * RPR + (RPR // CK) * CK, RPR % CK)])

    @pl.when(sid < NS - 1)
    def _():
        pltpu.sync_copy(y_hbm.at[pl.ds(sid * RPR, RPR)],
                        ytbl_sh.at[pl.ds(sid * RPR, RPR)])

    @pl.when(sid == NS - 1)
    def _():
        nlast = TR - (NS - 1) * RPR
        pltpu.sync_copy(y_hbm.at[pl.ds((NS - 1) * RPR, nlast)],
                        ytbl_sh.at[pl.ds((NS - 1) * RPR, nlast)])

    plsc.subcore_barrier()

    one = jnp.ones((16,), jnp.int32)
    zero16 = jnp.zeros((16,), jnp.float32)

    def run_layer():
        def blk_body(b, _):
            pltpu.sync_copy(src_hbm.at[sid, pl.ds(b * EB, EB)], src_v)
            pltpu.sync_copy(dst_hbm.at[sid, pl.ds(b * EB, EB)], dst_v)
            pltpu.sync_copy(ew_hbm.at[sid, pl.ds(b * EB * CK, EB * CK)], ew_v)

            def write_gidx(i, gref):
                def idx_group(g, _):
                    s16 = src_v[i, pl.ds(g * 16, 16)]
                    gref[0, pl.ds(g * 16, 16)] = lax.shift_right_logical(
                        s16, one)
                    return ()

                lax.fori_loop(0, CK // 16, idx_group, ())

            def scale_scatter(i, bref):
                def scale_group(g, _):
                    ew16 = ew_v[pl.ds(i * CK + g * 16, 16)]
                    ps16 = (src_v[i, pl.ds(g * 16, 16)] & one).astype(
                        jnp.float32)
                    d16 = dst_v[i, pl.ds(g * 16, 16)]
                    si_v[0, pl.ds(g * 16, 16)] = lax.shift_right_logical(
                        d16, one)
                    pd16 = (d16 & one).astype(jnp.float32)
                    for j in range(16):
                        r = g * 16 + j
                        w = _bcast_lane(ew16, j)
                        ps = _bcast_lane(ps16, j)
                        pd = _bcast_lane(pd16, j)
                        a = (1.0 - ps) * w
                        bb = ps * w
                        npd = 1.0 - pd
                        for q in range(H // 16):
                            lo = bref[r, pl.ds(q * 16, 16)]
                            hi = bref[r, pl.ds(H + q * 16, 16)]
                            val = lo * a + hi * bb
                            bref[r, pl.ds(q * 16, 16)] = val * npd
                            bref[r, pl.ds(H + q * 16, 16)] = val * pd
                    return ()

                lax.fori_loop(0, CK // 16, scale_group, ())
                pltpu.sync_copy(bref, acc_sh.at[si_v.at[0]], add=True)

            def pair_body(p, _):
                write_gidx(2 * p, gi_v)
                cpa = pltpu.async_copy(ytbl_sh.at[gi_v.at[0]], buf_v, sem0)
                write_gidx(2 * p + 1, gi2_v)
                cpb = pltpu.async_copy(ytbl_sh.at[gi2_v.at[0]], buf2_v, sem1)
                cpa.wait()
                scale_scatter(2 * p, buf_v)
                cpb.wait()
                scale_scatter(2 * p + 1, buf2_v)
                return ()

            lax.fori_loop(0, EB // 2, pair_body, ())
            return ()

        lax.fori_loop(0, CHL // EB, blk_body, ())

    run_layer()
    plsc.subcore_barrier()

    lax.fori_loop(0, CK, zero_buf, ())
    for blk in range(RPR // MB):
        base = sid * RPR + blk * MB
        pltpu.sync_copy(acc_sh.at[pl.ds(base, MB)], abuf_v)

        def row_group(g8, _):
            dv16 = dinv_v[pl.ds(blk * 2 * MB + g8 * 16, 16)]
            for j in range(8):
                r = g8 * 8 + j
                dlo = _bcast_lane(dv16, 2 * j)
                dhi = _bcast_lane(dv16, 2 * j + 1)
                for q in range(H // 16):
                    b1q = b1_v[pl.ds(q * 16, 16)]
                    lo = abuf_v[r, pl.ds(q * 16, 16)]
                    hi = abuf_v[r, pl.ds(H + q * 16, 16)]
                    abuf_v[r, pl.ds(q * 16, 16)] = dlo * jnp.maximum(
                        dlo * lo + b1q, 0.0)
                    abuf_v[r, pl.ds(H + q * 16, 16)] = dhi * jnp.maximum(
                        dhi * hi + b1q, 0.0)
            return ()

        lax.fori_loop(0, MB // 8, row_group, ())

        @pl.when(base + MB <= TR)
        def _():
            pltpu.sync_copy(abuf_v, ytbl_sh.at[pl.ds(base, MB)])

        pltpu.sync_copy(buf_v.at[pl.ds(0, MB)], acc_sh.at[pl.ds(base, MB)])

    plsc.subcore_barrier()

    run_layer()
    plsc.subcore_barrier()

    obase = cid * (NR // 2) + sid * (RPR // 2)
    pltpu.sync_copy(acc_sh.at[pl.ds(obase, RPR // 2)],
                    out_hbm.at[pl.ds(obase, RPR // 2)])


_BLK = 1280
_GRID = NP // _BLK


def _tc_pre_body(x_ref, w_ref, deg_ref, y_ref, dinv_ref):
    xw = jnp.dot(x_ref[...], w_ref[...], preferred_element_type=jnp.float32)
    deg = deg_ref[0] + deg_ref[1]
    dinv = jnp.where(deg > 0, lax.rsqrt(deg), 0.0)
    y_ref[...] = xw * dinv
    dinv_ref[...] = dinv


def _tc_fin_body(agg_ref, deg_ref, w2_ref, b_ref, wfc_ref, bfc_ref, o_ref):
    deg = deg_ref[0] + deg_ref[1]
    dinv = jnp.where(deg > 0, lax.rsqrt(deg), 0.0)
    a2 = jnp.dot(agg_ref[...], w2_ref[...], preferred_element_type=jnp.float32)
    h = jnp.maximum(a2 * dinv + b_ref[...], 0.0)
    mask = lax.broadcasted_iota(jnp.int32, (NP, 1), 0) < N
    h = jnp.where(mask, h, 0.0)
    g = jnp.sum(h, axis=0, keepdims=True) * (1.0 / N)
    o_ref[...] = jnp.dot(g, wfc_ref[...],
                         preferred_element_type=jnp.float32) + bfc_ref[...]


def _tc_pre(xp, W1, degp):
    return pl.pallas_call(
        _tc_pre_body,
        grid=(_GRID,),
        in_specs=[
            pl.BlockSpec((_BLK, D), lambda i: (i, 0)),
            pl.BlockSpec((D, H), lambda i: (0, 0)),
            pl.BlockSpec((NC, _BLK, 1), lambda i: (0, i, 0)),
        ],
        out_specs=[
            pl.BlockSpec((_BLK, H), lambda i: (i, 0)),
            pl.BlockSpec((_BLK, 1), lambda i: (i, 0)),
        ],
        out_shape=[
            jax.ShapeDtypeStruct((NP, H), jnp.float32),
            jax.ShapeDtypeStruct((NP, 1), jnp.float32),
        ],
    )(xp, W1, degp)


def _tc_fin(aggp, degp, W2, b2, Wfc, bfc):
    return pl.pallas_call(
        _tc_fin_body,
        in_specs=[
            pl.BlockSpec((NP, H), lambda: (0, 0)),
            pl.BlockSpec((NC, NP, 1), lambda: (0, 0, 0)),
            pl.BlockSpec((H, H), lambda: (0, 0)),
            pl.BlockSpec((1, H), lambda: (0, 0)),
            pl.BlockSpec((H, C), lambda: (0, 0)),
            pl.BlockSpec((1, C), lambda: (0, 0)),
        ],
        out_specs=pl.BlockSpec((1, C), lambda: (0, 0)),
        out_shape=jax.ShapeDtypeStruct((1, C), jnp.float32),
    )(aggp, degp, W2, b2, Wfc, bfc)


def kernel(x, edge_index, edge_attr, W1, b1, W2, b2, Wfc, bfc):
    loop = jnp.arange(N, dtype=jnp.int32)
    src = jnp.concatenate([edge_index[0], loop])
    dst = jnp.concatenate([edge_index[1], loop])
    ew = jnp.concatenate([edge_attr, jnp.ones((N,), jnp.float32)])

    padd = EPD - src.shape[0]
    dstd3 = jnp.concatenate(
        [dst, jnp.full((padd,), N, jnp.int32)]).reshape(NW, CHD, CK)
    ewd2 = jnp.concatenate(
        [ew, jnp.zeros((padd,), jnp.float32)]).reshape(NW, CHD * CK)

    padl = EPL - src.shape[0]
    srcl3 = jnp.pad(jnp.concatenate(
        [src, jnp.full((padl,), N, jnp.int32)]).reshape(NS, CHL, CK),
        ((0, 0), (0, CHP - CHL), (0, 0)))
    dstl3 = jnp.pad(jnp.concatenate(
        [dst, jnp.full((padl,), N, jnp.int32)]).reshape(NS, CHL, CK),
        ((0, 0), (0, CHP - CHL), (0, 0)))
    ewl2 = jnp.pad(jnp.concatenate(
        [ew, jnp.zeros((padl,), jnp.float32)]).reshape(NS, CHL * CK),
        ((0, 0), (0, (CHP - CHL) * CK)))

    xp = jnp.pad(x, ((0, NP - N), (0, 0)))
    zz = jnp.zeros((NP,), jnp.float32)

    degp = _deg_kernel(dstd3, ewd2, zz)
    degp3 = degp.reshape(NC, NP, 1)

    y1, dinv2 = _tc_pre(xp, W1, degp3)
    y1p = jnp.pad(y1.reshape(NR, 2 * H), ((0, YPAD - NR), (0, 0)))
    dinv1 = dinv2.reshape(NP)

    agg2p = _gcn_kernel(y1p, srcl3, dstl3, ewl2, dinv1, b1)
    agg2 = agg2p.reshape(NP, H)

    out = _tc_fin(agg2, degp3, W2, b2.reshape(1, H), Wfc, bfc.reshape(1, C))
    return out.reshape(C)


# speedup vs baseline: 1.0086x; 1.0086x over previous
import functools

import jax
import jax.numpy as jnp
from jax import lax
from jax.experimental import pallas as pl
from jax.experimental.pallas import tpu as pltpu
from jax.experimental.pallas import tpu_sc as plsc

N = 10000
D = 128
H = 64
C = 10

NC = 2
NS = 16
NW = NC * NS

NP = 10240
NR = NP // 2
RPR = NR // NS
TR = 5040
MB = 40
CK = 128
CHD = 82
CHL = 168
EB = 24
CHP = 600
YPAD = 16384
EPD = NW * CHD * CK
EPL = NS * CHL * CK

_mesh = plsc.VectorSubcoreMesh(core_axis_name="c", subcore_axis_name="s")


def _bcast_lane(vec16, j):
    idx = jnp.full((16,), j, dtype=jnp.int32)
    return lax.gather(
        vec16, idx[:, None],
        lax.GatherDimensionNumbers(
            offset_dims=(), collapsed_slice_dims=(0,), start_index_map=(0,)),
        (1,), mode=lax.GatherScatterMode.PROMISE_IN_BOUNDS)


@functools.partial(
    pl.kernel,
    out_type=jax.ShapeDtypeStruct((NC, NP), jnp.float32),
    mesh=_mesh,
    scratch_types=[
        pltpu.VMEM((CHD, CK), jnp.int32),
        pltpu.VMEM((CHD * CK,), jnp.float32),
        pltpu.VMEM_SHARED((NP,), jnp.float32),
    ],
)
def _deg_kernel(dst_hbm, ew_hbm, zz_hbm, out_hbm, dst_v, ew_v, acc_sh):
    cid = lax.axis_index("c")
    sid = lax.axis_index("s")
    wid = sid * NC + cid
    rpt = NP // NS
    pltpu.sync_copy(dst_hbm.at[wid], dst_v)
    pltpu.sync_copy(ew_hbm.at[wid], ew_v)
    pltpu.sync_copy(zz_hbm.at[pl.ds(sid * rpt, rpt)],
                    acc_sh.at[pl.ds(sid * rpt, rpt)])
    plsc.subcore_barrier()

    def body(i, _):
        pltpu.sync_copy(ew_v.at[pl.ds(i * CK, CK)],
                        acc_sh.at[dst_v.at[i]], add=True)
        return ()

    lax.fori_loop(0, CHD, body, ())
    plsc.subcore_barrier()
    pltpu.sync_copy(acc_sh.at[pl.ds(sid * rpt, rpt)],
                    out_hbm.at[cid, pl.ds(sid * rpt, rpt)])


@functools.partial(
    pl.kernel,
    out_type=jax.ShapeDtypeStruct((NR, 2 * H), jnp.float32),
    mesh=_mesh,
    scratch_types=[
        pltpu.VMEM((EB, CK), jnp.int32),
        pltpu.VMEM((EB, CK), jnp.int32),
        pltpu.VMEM((EB * CK,), jnp.float32),
        pltpu.VMEM((CK, 2 * H), jnp.float32),
        pltpu.VMEM((1, CK), jnp.int32),
        pltpu.VMEM((1, CK), jnp.int32),
        pltpu.VMEM((MB, 2 * H), jnp.float32),
        pltpu.VMEM((2 * RPR,), jnp.float32),
        pltpu.VMEM((H,), jnp.float32),
        pltpu.VMEM_SHARED((NR, 2 * H), jnp.float32),
        pltpu.VMEM_SHARED((TR, 2 * H), jnp.float32),
    ],
)
def _gcn_kernel(y_hbm, src_hbm, dst_hbm, ew_hbm, dinv_hbm, b1_hbm,
                out_hbm, src_v, dst_v, ew_v, buf_v, gi_v, si_v,
                abuf_v, dinv_v, b1_v, acc_sh, ytbl_sh):
    cid = lax.axis_index("c")
    sid = lax.axis_index("s")
    pltpu.sync_copy(dinv_hbm.at[pl.ds(sid * 2 * RPR, 2 * RPR)], dinv_v)
    pltpu.sync_copy(b1_hbm, b1_v)

    def zero_buf(_i, _):
        for q in range(2 * H // 16):
            buf_v[_i, pl.ds(q * 16, 16)] = jnp.zeros((16,), jnp.float32)
        return ()

    lax.fori_loop(0, CK, zero_buf, ())
    for k in range(RPR // CK):
        pltpu.sync_copy(buf_v, acc_sh.at[pl.ds(sid * RPR + k * CK, CK)])
    pltpu.sync_copy(buf_v.at[pl.ds(0, RPR % CK)],
                    acc_sh.at[pl.ds(sid * RPR + (RPR // CK) * CK, RPR % CK)])

    @pl.when(sid < NS - 1)
    def _():
        pltpu.sync_copy(y_hbm.at[pl.ds(sid * RPR, RPR)],
                        ytbl_sh.at[pl.ds(sid * RPR, RPR)])

    @pl.when(sid == NS - 1)
    def _():
        nlast = TR - (NS - 1) * RPR
        pltpu.sync_copy(y_hbm.at[pl.ds((NS - 1) * RPR, nlast)],
                        ytbl_sh.at[pl.ds((NS - 1) * RPR, nlast)])

    plsc.subcore_barrier()

    one = jnp.ones((16,), jnp.int32)
    zero16 = jnp.zeros((16,), jnp.float32)

    def run_layer():
        def blk_body(b, _):
            pltpu.sync_copy(src_hbm.at[sid, pl.ds(b * EB, EB)], src_v)
            pltpu.sync_copy(dst_hbm.at[sid, pl.ds(b * EB, EB)], dst_v)
            pltpu.sync_copy(ew_hbm.at[sid, pl.ds(b * EB * CK, EB * CK)], ew_v)

            def chunk_body(i, _):
                def idx_group(g, _):
                    s16 = src_v[i, pl.ds(g * 16, 16)]
                    d16 = dst_v[i, pl.ds(g * 16, 16)]
                    gi_v[0, pl.ds(g * 16, 16)] = lax.shift_right_logical(
                        s16, one)
                    si_v[0, pl.ds(g * 16, 16)] = lax.shift_right_logical(
                        d16, one)
                    return ()

                lax.fori_loop(0, CK // 16, idx_group, ())
                pltpu.sync_copy(ytbl_sh.at[gi_v.at[0]], buf_v)

                def scale_group(g, _):
                    ew16 = ew_v[pl.ds(i * CK + g * 16, 16)]
                    ps16 = (src_v[i, pl.ds(g * 16, 16)] & one).astype(
                        jnp.float32)
                    pd16 = (dst_v[i, pl.ds(g * 16, 16)] & one).astype(
                        jnp.float32)
                    for j in range(16):
                        r = g * 16 + j
                        w = _bcast_lane(ew16, j)
                        ps = _bcast_lane(ps16, j)
                        pd = _bcast_lane(pd16, j)
                        nps = 1.0 - ps
                        npd = 1.0 - pd
                        for q in range(H // 16):
                            lo = buf_v[r, pl.ds(q * 16, 16)]
                            hi = buf_v[r, pl.ds(H + q * 16, 16)]
                            val = (lo * nps + hi * ps) * w
                            buf_v[r, pl.ds(q * 16, 16)] = val * npd
                            buf_v[r, pl.ds(H + q * 16, 16)] = val * pd
                    return ()

                lax.fori_loop(0, CK // 16, scale_group, ())
                pltpu.sync_copy(buf_v, acc_sh.at[si_v.at[0]], add=True)
                return ()

            lax.fori_loop(0, EB, chunk_body, ())
            return ()

        lax.fori_loop(0, CHL // EB, blk_body, ())

    run_layer()
    plsc.subcore_barrier()

    lax.fori_loop(0, CK, zero_buf, ())
    for blk in range(RPR // MB):
        base = sid * RPR + blk * MB
        pltpu.sync_copy(acc_sh.at[pl.ds(base, MB)], abuf_v)

        def row_group(g8, _):
            dv16 = dinv_v[pl.ds(blk * 2 * MB + g8 * 16, 16)]
            for j in range(8):
                r = g8 * 8 + j
                dlo = _bcast_lane(dv16, 2 * j)
                dhi = _bcast_lane(dv16, 2 * j + 1)
                for q in range(H // 16):
                    b1q = b1_v[pl.ds(q * 16, 16)]
                    lo = abuf_v[r, pl.ds(q * 16, 16)]
                    hi = abuf_v[r, pl.ds(H + q * 16, 16)]
                    abuf_v[r, pl.ds(q * 16, 16)] = dlo * jnp.maximum(
                        dlo * lo + b1q, 0.0)
                    abuf_v[r, pl.ds(H + q * 16, 16)] = dhi * jnp.maximum(
                        dhi * hi + b1q, 0.0)
            return ()

        lax.fori_loop(0, MB // 8, row_group, ())

        @pl.when(base + MB <= TR)
        def _():
            pltpu.sync_copy(abuf_v, ytbl_sh.at[pl.ds(base, MB)])

        pltpu.sync_copy(buf_v.at[pl.ds(0, MB)], acc_sh.at[pl.ds(base, MB)])

    plsc.subcore_barrier()

    run_layer()
    plsc.subcore_barrier()

    obase = cid * (NR // 2) + sid * (RPR // 2)
    pltpu.sync_copy(acc_sh.at[pl.ds(obase, RPR // 2)],
                    out_hbm.at[pl.ds(obase, RPR // 2)])


_BLK = 1280
_GRID = NP // _BLK


def _tc_pre_body(x_ref, w_ref, deg_ref, y_ref, dinv_ref):
    xw = jnp.dot(x_ref[...], w_ref[...], preferred_element_type=jnp.float32)
    deg = deg_ref[0] + deg_ref[1]
    dinv = jnp.where(deg > 0, lax.rsqrt(deg), 0.0)
    y_ref[...] = xw * dinv
    dinv_ref[...] = dinv


def _tc_fin_body(agg_ref, deg_ref, w2_ref, b_ref, wfc_ref, bfc_ref, o_ref):
    deg = deg_ref[0] + deg_ref[1]
    dinv = jnp.where(deg > 0, lax.rsqrt(deg), 0.0)
    a2 = jnp.dot(agg_ref[...], w2_ref[...], preferred_element_type=jnp.float32)
    h = jnp.maximum(a2 * dinv + b_ref[...], 0.0)
    mask = lax.broadcasted_iota(jnp.int32, (NP, 1), 0) < N
    h = jnp.where(mask, h, 0.0)
    g = jnp.sum(h, axis=0, keepdims=True) * (1.0 / N)
    o_ref[...] = jnp.dot(g, wfc_ref[...],
                         preferred_element_type=jnp.float32) + bfc_ref[...]


def _tc_pre(xp, W1, degp):
    return pl.pallas_call(
        _tc_pre_body,
        grid=(_GRID,),
        in_specs=[
            pl.BlockSpec((_BLK, D), lambda i: (i, 0)),
            pl.BlockSpec((D, H), lambda i: (0, 0)),
            pl.BlockSpec((NC, _BLK, 1), lambda i: (0, i, 0)),
        ],
        out_specs=[
            pl.BlockSpec((_BLK, H), lambda i: (i, 0)),
            pl.BlockSpec((_BLK, 1), lambda i: (i, 0)),
        ],
        out_shape=[
            jax.ShapeDtypeStruct((NP, H), jnp.float32),
            jax.ShapeDtypeStruct((NP, 1), jnp.float32),
        ],
    )(xp, W1, degp)


def _tc_fin(aggp, degp, W2, b2, Wfc, bfc):
    return pl.pallas_call(
        _tc_fin_body,
        in_specs=[
            pl.BlockSpec((NP, H), lambda: (0, 0)),
            pl.BlockSpec((NC, NP, 1), lambda: (0, 0, 0)),
            pl.BlockSpec((H, H), lambda: (0, 0)),
            pl.BlockSpec((1, H), lambda: (0, 0)),
            pl.BlockSpec((H, C), lambda: (0, 0)),
            pl.BlockSpec((1, C), lambda: (0, 0)),
        ],
        out_specs=pl.BlockSpec((1, C), lambda: (0, 0)),
        out_shape=jax.ShapeDtypeStruct((1, C), jnp.float32),
    )(aggp, degp, W2, b2, Wfc, bfc)


def kernel(x, edge_index, edge_attr, W1, b1, W2, b2, Wfc, bfc):
    loop = jnp.arange(N, dtype=jnp.int32)
    src = jnp.concatenate([edge_index[0], loop])
    dst = jnp.concatenate([edge_index[1], loop])
    ew = jnp.concatenate([edge_attr, jnp.ones((N,), jnp.float32)])

    padd = EPD - src.shape[0]
    dstd3 = jnp.concatenate(
        [dst, jnp.full((padd,), N, jnp.int32)]).reshape(NW, CHD, CK)
    ewd2 = jnp.concatenate(
        [ew, jnp.zeros((padd,), jnp.float32)]).reshape(NW, CHD * CK)

    padl = EPL - src.shape[0]
    srcl3 = jnp.pad(jnp.concatenate(
        [src, jnp.full((padl,), N, jnp.int32)]).reshape(NS, CHL, CK),
        ((0, 0), (0, CHP - CHL), (0, 0)))
    dstl3 = jnp.pad(jnp.concatenate(
        [dst, jnp.full((padl,), N, jnp.int32)]).reshape(NS, CHL, CK),
        ((0, 0), (0, CHP - CHL), (0, 0)))
    ewl2 = jnp.pad(jnp.concatenate(
        [ew, jnp.zeros((padl,), jnp.float32)]).reshape(NS, CHL * CK),
        ((0, 0), (0, (CHP - CHL) * CK)))

    xp = jnp.pad(x, ((0, NP - N), (0, 0)))
    zz = jnp.zeros((NP,), jnp.float32)

    degp = _deg_kernel(dstd3, ewd2, zz)
    degp3 = degp.reshape(NC, NP, 1)

    y1, dinv2 = _tc_pre(xp, W1, degp3)
    y1p = jnp.pad(y1.reshape(NR, 2 * H), ((0, YPAD - NR), (0, 0)))
    dinv1 = dinv2.reshape(NP)

    agg2p = _gcn_kernel(y1p, srcl3, dstl3, ewl2, dinv1, b1)
    agg2 = agg2p.reshape(NP, H)

    out = _tc_fin(agg2, degp3, W2, b2.reshape(1, H), Wfc, bfc.reshape(1, C))
    return out.reshape(C)
